# baseline (speedup 1.0000x reference)
import functools

import jax
import jax.numpy as jnp
from jax import lax
from jax.experimental import pallas as pl
from jax.experimental.pallas import tpu as pltpu

_LANES = 128
_SUBL = 8


def _count_body(predT, g_smem, g_ref, out1_ref, out5_ref,
                bufs, twin, rowbuf, sems, tsem,
                *, n, bsz, cr, nbuf):
    nchunks = n // cr
    ngrp = bsz // _LANES

    rowbuf[...] = lax.broadcasted_iota(jnp.int32, (cr, bsz), 0)

    def win_copy(i):
        r0 = pl.multiple_of((g_smem[i] // _SUBL) * _SUBL, _SUBL)
        grp = i // _LANES
        lane = lax.rem(i, _LANES)
        c0 = pl.multiple_of((i // _LANES) * _LANES, _LANES)
        return pltpu.make_async_copy(
            predT.at[pl.ds(r0, _SUBL), pl.ds(c0, _LANES)],
            twin.at[grp, pl.ds(lane * _SUBL, _SUBL), :],
            tsem,
        )

    wave = _LANES
    for w in range(bsz // wave):
        lax.fori_loop(w * wave, (w + 1) * wave,
                      lambda i, _: (win_copy(i).start(), 0)[1], 0)
        lax.fori_loop(w * wave, (w + 1) * wave,
                      lambda i, _: (win_copy(i).wait(), 0)[1], 0)

    def issue(c, b):
        pltpu.make_async_copy(
            predT.at[pl.ds(c * cr, cr), :],
            bufs.at[pl.ds(b * cr, cr), :], sems.at[b]
        ).start()

    for b in range(min(nbuf, nchunks)):
        issue(b, b)

    g = g_ref[...]
    pieces = []
    sub_i = lax.broadcasted_iota(jnp.int32, (_SUBL * _LANES, _LANES), 0)
    lane_i = lax.broadcasted_iota(jnp.int32, (_SUBL * _LANES, _LANES), 1)
    for grp in range(ngrp):
        a = twin[grp]
        gmod = g[:, grp * _LANES:(grp + 1) * _LANES] % _SUBL
        sel = sub_i == (lane_i * _SUBL + gmod)
        pieces.append(jnp.sum(jnp.where(sel, a, 0.0), axis=0, keepdims=True))
    t = jnp.concatenate(pieces, axis=1)

    def step(c, carry):
        acc = carry
        b = lax.rem(c, nbuf)
        pltpu.make_async_copy(
            predT.at[pl.ds(c * cr, cr), :],
            bufs.at[pl.ds(b * cr, cr), :], sems.at[b]
        ).wait()
        p = bufs[pl.ds(b * cr, cr), :]
        gl = g - c * cr
        row = rowbuf[...]
        ahead = (p > t) | ((p == t) & (row < gl))
        acc += jnp.sum(ahead.astype(jnp.float32), axis=0, keepdims=True)

        nc = c + nbuf

        @pl.when(nc < nchunks)
        def _refill():
            issue(nc, b)

        return acc

    rank = lax.fori_loop(0, nchunks, step,
                         jnp.zeros((1, bsz), jnp.float32))
    c1 = jnp.sum((rank < 1.0).astype(jnp.float32), axis=1, keepdims=True)
    c5 = jnp.sum((rank < 5.0).astype(jnp.float32), axis=1, keepdims=True)
    out1_ref[...] = c1 * (100.0 / bsz)
    out5_ref[...] = c5 * (100.0 / bsz)


def _count(predT, g_flat, g_row, *, cr, nbuf=8, interpret=False):
    N, B = predT.shape
    body = functools.partial(_count_body, n=N, bsz=B, cr=cr, nbuf=nbuf)
    return pl.pallas_call(
        body,
        in_specs=[
            pl.BlockSpec(memory_space=pltpu.MemorySpace.HBM),
            pl.BlockSpec(memory_space=pltpu.MemorySpace.SMEM),
            pl.BlockSpec((1, B), lambda: (0, 0)),
        ],
        out_specs=[
            pl.BlockSpec((1, 1), lambda: (0, 0)),
            pl.BlockSpec((1, 1), lambda: (0, 0)),
        ],
        out_shape=[
            jax.ShapeDtypeStruct((1, 1), jnp.float32),
            jax.ShapeDtypeStruct((1, 1), jnp.float32),
        ],
        scratch_shapes=[
            pltpu.VMEM((nbuf * cr, B), jnp.float32),
            pltpu.VMEM((B // _LANES, _SUBL * _LANES, _LANES), jnp.float32),
            pltpu.VMEM((cr, B), jnp.int32),
            pltpu.SemaphoreType.DMA((nbuf,)),
            pltpu.SemaphoreType.DMA,
        ],
        interpret=interpret,
    )(predT, g_flat, g_row)


def kernel(pred, target):
    B, N = pred.shape
    out1, out5 = _count(pred.T, target, target.reshape(1, B), cr=1000)
    return (out1.reshape(1), out5.reshape(1))


# speedup vs baseline: 1.0164x; 1.0164x over previous
import functools

import jax
import jax.numpy as jnp
from jax import lax
from jax.experimental import pallas as pl
from jax.experimental.pallas import tpu as pltpu

_LANES = 128
_SUBL = 8


def _count_body(predT, g_smem, g_ref, out1_ref, out5_ref,
                bufs, twin, rowbuf, sems, tsem,
                *, n, bsz, cr, nbuf):
    nchunks = n // cr
    ngrp = bsz // _LANES

    rowbuf[...] = lax.broadcasted_iota(jnp.int32, (cr, bsz), 0)

    def win_copy(i):
        r0 = pl.multiple_of((g_smem[i] // _SUBL) * _SUBL, _SUBL)
        grp = i // _LANES
        lane = lax.rem(i, _LANES)
        c0 = pl.multiple_of((i // _LANES) * _LANES, _LANES)
        return pltpu.make_async_copy(
            predT.at[pl.ds(r0, _SUBL), pl.ds(c0, _LANES)],
            twin.at[grp, pl.ds(lane * _SUBL, _SUBL), :],
            tsem,
        )

    wave = _LANES
    for w in range(bsz // wave):
        lax.fori_loop(w * wave, (w + 1) * wave,
                      lambda i, _: (win_copy(i).start(), 0)[1], 0)
        lax.fori_loop(w * wave, (w + 1) * wave,
                      lambda i, _: (win_copy(i).wait(), 0)[1], 0)

    def issue(c, b):
        pltpu.make_async_copy(
            predT.at[pl.ds(c * cr, cr), :], bufs.at[b], sems.at[b]
        ).start()

    for b in range(min(nbuf, nchunks)):
        issue(b, b)

    g = g_ref[...]
    pieces = []
    sub_i = lax.broadcasted_iota(jnp.int32, (_SUBL * _LANES, _LANES), 0)
    lane_i = lax.broadcasted_iota(jnp.int32, (_SUBL * _LANES, _LANES), 1)
    for grp in range(ngrp):
        a = twin[grp]
        gmod = g[:, grp * _LANES:(grp + 1) * _LANES] % _SUBL
        sel = sub_i == (lane_i * _SUBL + gmod)
        pieces.append(jnp.sum(jnp.where(sel, a, 0.0), axis=0, keepdims=True))
    t = jnp.concatenate(pieces, axis=1)

    def round_step(rd, carry):
        acc = carry
        base = rd * nbuf
        for k in range(nbuf):
            c = base + k
            pltpu.make_async_copy(
                predT.at[pl.ds(c * cr, cr), :], bufs.at[k], sems.at[k]
            ).wait()
            p = bufs[k]
            gl = g - c * cr
            row = rowbuf[...]
            ahead = (p > t) | ((p == t) & (row < gl))
            acc += jnp.sum(ahead.astype(jnp.float32), axis=0, keepdims=True)

            nc = c + nbuf

            @pl.when(nc < nchunks)
            def _refill():
                issue(nc, k)

        return acc

    rank = lax.fori_loop(0, nchunks // nbuf, round_step,
                         jnp.zeros((1, bsz), jnp.float32))
    for k in range(nchunks % nbuf):
        c = (nchunks // nbuf) * nbuf + k
        pltpu.make_async_copy(
            predT.at[pl.ds(c * cr, cr), :], bufs.at[k], sems.at[k]
        ).wait()
        p = bufs[k]
        gl = g - c * cr
        row = rowbuf[...]
        ahead = (p > t) | ((p == t) & (row < gl))
        rank += jnp.sum(ahead.astype(jnp.float32), axis=0, keepdims=True)
    c1 = jnp.sum((rank < 1.0).astype(jnp.float32), axis=1, keepdims=True)
    c5 = jnp.sum((rank < 5.0).astype(jnp.float32), axis=1, keepdims=True)
    out1_ref[...] = c1 * (100.0 / bsz)
    out5_ref[...] = c5 * (100.0 / bsz)


def _count(predT, g_flat, g_row, *, cr, nbuf=4, interpret=False):
    N, B = predT.shape
    body = functools.partial(_count_body, n=N, bsz=B, cr=cr, nbuf=nbuf)
    return pl.pallas_call(
        body,
        in_specs=[
            pl.BlockSpec(memory_space=pltpu.MemorySpace.HBM),
            pl.BlockSpec(memory_space=pltpu.MemorySpace.SMEM),
            pl.BlockSpec((1, B), lambda: (0, 0)),
        ],
        out_specs=[
            pl.BlockSpec((1, 1), lambda: (0, 0)),
            pl.BlockSpec((1, 1), lambda: (0, 0)),
        ],
        out_shape=[
            jax.ShapeDtypeStruct((1, 1), jnp.float32),
            jax.ShapeDtypeStruct((1, 1), jnp.float32),
        ],
        scratch_shapes=[
            pltpu.VMEM((nbuf, cr, B), jnp.float32),
            pltpu.VMEM((B // _LANES, _SUBL * _LANES, _LANES), jnp.float32),
            pltpu.VMEM((cr, B), jnp.int32),
            pltpu.SemaphoreType.DMA((nbuf,)),
            pltpu.SemaphoreType.DMA,
        ],
        interpret=interpret,
    )(predT, g_flat, g_row)


def kernel(pred, target):
    B, N = pred.shape
    out1, out5 = _count(pred.T, target, target.reshape(1, B), cr=1000)
    return (out1.reshape(1), out5.reshape(1))
